# baseline (speedup 1.0000x reference)
import functools

import jax
import jax.numpy as jnp
from jax import lax
from jax.experimental import pallas as pl
from jax.experimental.pallas import tpu as pltpu
from jax.experimental.pallas import tpu_sc as plsc

CLS_NUM = 100000
FEATURE_NUM = 64
BATCH = 16384

NC = 2
NS = 16
NW = NC * NS
B_PER_W = BATCH // NW
GROUPS = B_PER_W // 16
HIST_PAD = 100096


def _sc_body(labels_hbm, datas_hbm, center_hbm, d2_hbm, cnt_hbm,
             labv_my, labv_hist, zeros_v, ones_v, cntv, d2_v,
             rows_v, datas_v, hist, sem):
    cid = lax.axis_index("c")
    sid = lax.axis_index("s")
    wid = sid * NC + cid

    pltpu.sync_copy(labels_hbm.at[pl.ds(wid * 4, 4)], labv_my)
    cps = [
        pltpu.async_copy(
            center_hbm.at[labv_my.at[k]],
            rows_v.at[pl.ds(k * 128, 128)],
            sem,
        )
        for k in range(4)
    ]

    for j in range(8):
        zeros_v[pl.ds(j * 16, 16)] = jnp.zeros((16,), jnp.float32)
        ones_v[pl.ds(j * 16, 16)] = jnp.ones((16,), jnp.float32)

    pltpu.sync_copy(labels_hbm.at[pl.ds(sid * 8, 8)], labv_hist)

    for k in range(8):
        pltpu.sync_copy(zeros_v, hist.at[labv_hist.at[k]])
    plsc.subcore_barrier()
    for k in range(8):
        pltpu.sync_copy(ones_v, hist.at[labv_hist.at[k]], add=True)

    pltpu.sync_copy(datas_hbm.at[pl.ds(wid * 256, 256)], datas_v)

    for cp in cps:
        cp.wait()

    iota = lax.iota(jnp.int32, 16)
    half = (iota & 1) * 64

    def group(g, _):
        svec = g * 16 + iota
        ridx_r = svec
        ridx_d = lax.shift_right_logical(svec, 1)
        acc = [jnp.zeros((16,), jnp.float32) for _ in range(4)]
        for f in range(FEATURE_NUM):
            a = plsc.load_gather(datas_v, [ridx_d, half + f])
            b = plsc.load_gather(rows_v, [ridx_r, jnp.full((16,), f, jnp.int32)])
            d = a - b
            acc[f % 4] = acc[f % 4] + d * d
        tot = (acc[0] + acc[1]) + (acc[2] + acc[3])
        d2_v[pl.ds(pl.multiple_of(g * 16, 16), 16)] = tot
        return _

    lax.fori_loop(0, GROUPS, group, 0)
    pltpu.sync_copy(d2_v, d2_hbm.at[pl.ds(wid * B_PER_W, B_PER_W)])

    plsc.subcore_barrier()

    for k in range(4):
        pltpu.sync_copy(hist.at[labv_my.at[k]], cntv.at[pl.ds(k * 128, 128)])
    pltpu.sync_copy(cntv, cnt_hbm.at[pl.ds(wid * B_PER_W, B_PER_W)])


_sc_main = functools.partial(
    pl.kernel,
    mesh=plsc.VectorSubcoreMesh(core_axis_name="c", subcore_axis_name="s"),
    compiler_params=pltpu.CompilerParams(
        use_tc_tiling_on_sc=False, needs_layout_passes=False
    ),
    out_type=[
        jax.ShapeDtypeStruct((BATCH,), jnp.float32),
        jax.ShapeDtypeStruct((BATCH,), jnp.float32),
    ],
    scratch_types=[
        pltpu.VMEM((4, 128), jnp.int32),
        pltpu.VMEM((8, 128), jnp.int32),
        pltpu.VMEM((128,), jnp.float32),
        pltpu.VMEM((128,), jnp.float32),
        pltpu.VMEM((B_PER_W,), jnp.float32),
        pltpu.VMEM((B_PER_W,), jnp.float32),
        pltpu.VMEM((B_PER_W, FEATURE_NUM), jnp.float32),
        pltpu.VMEM((B_PER_W // 2, 128), jnp.float32),
        pltpu.VMEM_SHARED((HIST_PAD,), jnp.float32),
        pltpu.SemaphoreType.DMA,
    ],
)(_sc_body)


def _tc_body(d2_ref, cnt_ref, out_ref):
    out_ref[...] = jnp.sum(jnp.sqrt(d2_ref[...]) / cnt_ref[...]).reshape(1, 1)


_tc_tail = pl.pallas_call(
    _tc_body,
    out_shape=jax.ShapeDtypeStruct((1, 1), jnp.float32),
)


@jax.jit
def kernel(datas, labels, center):
    labels2d = labels.astype(jnp.int32).reshape(BATCH // 128, 128)
    datas_flat = datas.reshape(BATCH // 2, 128)
    d2, cnt = _sc_main(labels2d, datas_flat, center)
    out = _tc_tail(d2.reshape(128, 128), cnt.reshape(128, 128))
    return out[0, 0]


# speedup vs baseline: 1.2400x; 1.2400x over previous
import functools

import jax
import jax.numpy as jnp
from jax import lax
from jax.experimental import pallas as pl
from jax.experimental.pallas import tpu as pltpu
from jax.experimental.pallas import tpu_sc as plsc

CLS_NUM = 100000
FEATURE_NUM = 64
BATCH = 16384

NC = 2
NS = 16
NW = NC * NS
B_PER_W = BATCH // NW
R_PER_W = B_PER_W // 2
HIST_PAD = 100096


def _sc_body(labels_hbm, datas_hbm, center_hbm, rows_hbm, datas_out,
             cnt_hbm, labv_my, labv_hist, zeros_v, ones_v, cntv, rows_v,
             datas_v, cnt_pad, hist, sem):
    cid = lax.axis_index("c")
    sid = lax.axis_index("s")
    wid = sid * NC + cid

    pltpu.sync_copy(labels_hbm.at[pl.ds(wid * 4, 4)], labv_my)
    cps = [
        pltpu.async_copy(
            center_hbm.at[labv_my.at[k]],
            rows_v.at[pl.ds(k * 128, 128)],
            sem,
        )
        for k in range(4)
    ]

    for j in range(8):
        zeros_v[pl.ds(j * 16, 16)] = jnp.zeros((16,), jnp.float32)
        ones_v[pl.ds(j * 16, 16)] = jnp.ones((16,), jnp.float32)

    pltpu.sync_copy(labels_hbm.at[pl.ds(sid * 8, 8)], labv_hist)

    for k in range(8):
        pltpu.sync_copy(zeros_v, hist.at[labv_hist.at[k]])
    plsc.subcore_barrier()
    for k in range(8):
        pltpu.sync_copy(ones_v, hist.at[labv_hist.at[k]], add=True)
    plsc.subcore_barrier()

    for k in range(4):
        pltpu.sync_copy(hist.at[labv_my.at[k]], cntv.at[pl.ds(k * 128, 128)])
    iota = lax.iota(jnp.int32, 16)
    czero = jnp.zeros((16,), jnp.int32)
    cone = jnp.full((16,), 1, jnp.int32)
    for g in range(R_PER_W // 16):
        ridx = g * 16 + iota
        ce = plsc.load_gather(cntv, [g * 32 + 2 * iota])
        co = plsc.load_gather(cntv, [g * 32 + 2 * iota + 1])
        plsc.store_scatter(cnt_pad, [ridx, czero], ce)
        plsc.store_scatter(cnt_pad, [ridx, cone], co)
    pltpu.sync_copy(cnt_pad, cnt_hbm.at[pl.ds(wid * R_PER_W, R_PER_W)])

    pltpu.sync_copy(datas_hbm.at[pl.ds(wid * B_PER_W, B_PER_W)], datas_v)
    pltpu.sync_copy(datas_v, datas_out.at[pl.ds(wid * B_PER_W, B_PER_W)])

    for cp in cps:
        cp.wait()
    pltpu.sync_copy(rows_v, rows_hbm.at[pl.ds(wid * B_PER_W, B_PER_W)])


_sc_gather = functools.partial(
    pl.kernel,
    mesh=plsc.VectorSubcoreMesh(core_axis_name="c", subcore_axis_name="s"),
    compiler_params=pltpu.CompilerParams(
        use_tc_tiling_on_sc=False, needs_layout_passes=False
    ),
    out_type=[
        jax.ShapeDtypeStruct((BATCH, FEATURE_NUM), jnp.float32),
        jax.ShapeDtypeStruct((BATCH, FEATURE_NUM), jnp.float32),
        jax.ShapeDtypeStruct((BATCH // 2, 128), jnp.float32),
    ],
    scratch_types=[
        pltpu.VMEM((4, 128), jnp.int32),
        pltpu.VMEM((8, 128), jnp.int32),
        pltpu.VMEM((128,), jnp.float32),
        pltpu.VMEM((128,), jnp.float32),
        pltpu.VMEM((B_PER_W,), jnp.float32),
        pltpu.VMEM((B_PER_W, FEATURE_NUM), jnp.float32),
        pltpu.VMEM((B_PER_W, FEATURE_NUM), jnp.float32),
        pltpu.VMEM((R_PER_W, 128), jnp.float32),
        pltpu.VMEM_SHARED((HIST_PAD,), jnp.float32),
        pltpu.SemaphoreType.DMA,
    ],
)(_sc_body)


def _tc_body(datas_ref, rows_ref, cnt_ref, out_ref):
    diff = datas_ref[...] - rows_ref[...]
    sq = diff * diff
    d2e = jnp.sum(sq[:, :FEATURE_NUM], axis=1, keepdims=True)
    d2o = jnp.sum(sq[:, FEATURE_NUM:], axis=1, keepdims=True)
    ce = cnt_ref[:, 0:1]
    co = cnt_ref[:, 1:2]
    tot = jnp.sum(jnp.sqrt(d2e) / ce) + jnp.sum(jnp.sqrt(d2o) / co)
    out_ref[...] = tot.reshape(1, 1)


_tc_tail = pl.pallas_call(
    _tc_body,
    out_shape=jax.ShapeDtypeStruct((1, 1), jnp.float32),
)


@jax.jit
def kernel(datas, labels, center):
    labels2d = labels.astype(jnp.int32).reshape(BATCH // 128, 128)
    rows, datas_lin, cnt_pad = _sc_gather(labels2d, datas, center)
    out = _tc_tail(
        datas_lin.reshape(BATCH // 2, 128),
        rows.reshape(BATCH // 2, 128),
        cnt_pad,
    )
    return out[0, 0]


# speedup vs baseline: 1.2530x; 1.0105x over previous
import functools

import jax
import jax.numpy as jnp
from jax import lax
from jax.experimental import pallas as pl
from jax.experimental.pallas import tpu as pltpu
from jax.experimental.pallas import tpu_sc as plsc

CLS_NUM = 100000
FEATURE_NUM = 64
BATCH = 16384

NC = 2
NS = 16
NW = NC * NS
B_PER_W = BATCH // NW
HIST_PAD = 100096


def _sc_body(labels_hbm, center_hbm, out_hbm,
             labv_my, labv_hist, zeros_v, ones_v, cntv, rows_v, rows_vp,
             hist, sem):
    cid = lax.axis_index("c")
    sid = lax.axis_index("s")
    wid = sid * NC + cid

    pltpu.sync_copy(labels_hbm.at[pl.ds(wid * 4, 4)], labv_my)
    cps = [
        pltpu.async_copy(
            center_hbm.at[labv_my.at[k]],
            rows_v.at[pl.ds(k * 128, 128)],
            sem,
        )
        for k in range(4)
    ]

    for j in range(8):
        zeros_v[pl.ds(j * 16, 16)] = jnp.zeros((16,), jnp.float32)
        ones_v[pl.ds(j * 16, 16)] = jnp.ones((16,), jnp.float32)

    pltpu.sync_copy(labels_hbm.at[pl.ds(sid * 8, 8)], labv_hist)

    for k in range(8):
        pltpu.sync_copy(zeros_v, hist.at[labv_hist.at[k]])
    plsc.subcore_barrier()
    for k in range(8):
        pltpu.sync_copy(ones_v, hist.at[labv_hist.at[k]], add=True)
    plsc.subcore_barrier()

    for k in range(4):
        pltpu.sync_copy(hist.at[labv_my.at[k]], cntv.at[pl.ds(k * 128, 128)])

    for cp in cps:
        cp.wait()

    def repack(i, carry):
        base = pl.multiple_of(i * 4, 4)
        for su in range(4):
            for q in range(4):
                rows_vp[base + su, pl.ds(q * 16, 16)] = (
                    rows_v[base + su, pl.ds(q * 16, 16)]
                )
        return carry

    lax.fori_loop(0, B_PER_W // 4, repack, 0)

    iota = lax.iota(jnp.int32, 16)
    c64 = jnp.full((16,), FEATURE_NUM, jnp.int32)
    for g in range(B_PER_W // 16):
        cv = cntv[pl.ds(g * 16, 16)]
        plsc.store_scatter(rows_vp, [g * 16 + iota, c64], cv)

    pltpu.sync_copy(rows_vp, out_hbm.at[pl.ds(wid * B_PER_W, B_PER_W)])


_sc_gather = functools.partial(
    pl.kernel,
    mesh=plsc.VectorSubcoreMesh(core_axis_name="c", subcore_axis_name="s"),
    compiler_params=pltpu.CompilerParams(
        use_tc_tiling_on_sc=False, needs_layout_passes=False
    ),
    out_type=[
        jax.ShapeDtypeStruct((BATCH, 128), jnp.float32),
    ],
    scratch_types=[
        pltpu.VMEM((4, 128), jnp.int32),
        pltpu.VMEM((8, 128), jnp.int32),
        pltpu.VMEM((128,), jnp.float32),
        pltpu.VMEM((128,), jnp.float32),
        pltpu.VMEM((B_PER_W,), jnp.float32),
        pltpu.VMEM((B_PER_W, FEATURE_NUM), jnp.float32),
        pltpu.VMEM((B_PER_W, 128), jnp.float32),
        pltpu.VMEM_SHARED((HIST_PAD,), jnp.float32),
        pltpu.SemaphoreType.DMA,
    ],
)(_sc_body)


def _tc_body(datas_ref, slab_ref, out_ref):
    x = datas_ref[...]
    slab = slab_ref[...]
    diff = x - slab[:, :FEATURE_NUM]
    d2 = jnp.sum(diff * diff, axis=1, keepdims=True)
    cnt = slab[:, FEATURE_NUM:FEATURE_NUM + 1]
    out_ref[...] = jnp.sum(jnp.sqrt(d2) / cnt).reshape(1, 1)


_tc_tail = pl.pallas_call(
    _tc_body,
    out_shape=jax.ShapeDtypeStruct((1, 1), jnp.float32),
)


@jax.jit
def kernel(datas, labels, center):
    labels2d = labels.astype(jnp.int32).reshape(BATCH // 128, 128)
    (slab,) = _sc_gather(labels2d, center)
    out = _tc_tail(datas, slab)
    return out[0, 0]
